# Initial kernel scaffold; baseline (speedup 1.0000x reference)
import jax
import jax.numpy as jnp
from jax.experimental import pallas as pl


def kernel(x, edge_index, edge_attr, batch_idx, Wa, ba, Wb, bb, Wm0, bm0, gm0, betam0, Wu0, bu0, gu0, betau0, Wm1, bm1, gm1, betam1, Wu1, bu1, gu1, betau1, W1, b1, W2, b2):
    raise NotImplementedError("write your pallas kernel here")


# speedup vs baseline: 3.5617x; 3.5617x over previous
import functools

import jax
import jax.numpy as jnp
from jax import lax
from jax.experimental import pallas as pl
from jax.experimental.pallas import tpu as pltpu
from jax.experimental.pallas import tpu_sc as plsc

D = 128
G = 256
EPS = 1e-5

NC = 2
NS = 16
NW = NC * NS
CHUNK = 128


def _ln_tanh(z, g, b):
    mu = jnp.mean(z, axis=-1, keepdims=True)
    var = jnp.mean((z - mu) ** 2, axis=-1, keepdims=True)
    return jnp.tanh((z - mu) * jax.lax.rsqrt(var + EPS) * g + b)


def _full(shape):
    return pl.BlockSpec(shape, lambda i: tuple(0 for _ in shape))


def _node0(x, Wa, ba, WmA0, WmB0):
    N = x.shape[0]
    R = 2000
    def body(x_r, Wa_r, ba_r, WmA_r, WmB_r, h_r, a_r, b_r):
        h = jnp.dot(x_r[...], Wa_r[...], preferred_element_type=jnp.float32) + ba_r[...]
        h_r[...] = h
        a_r[...] = jnp.dot(h, WmA_r[...], preferred_element_type=jnp.float32)
        b_r[...] = jnp.dot(h, WmB_r[...], preferred_element_type=jnp.float32)
    out = jax.ShapeDtypeStruct((N, D), jnp.float32)
    return pl.pallas_call(
        body,
        grid=(N // R,),
        in_specs=[pl.BlockSpec((R, D), lambda i: (i, 0)), _full((D, D)),
                  _full((1, D)), _full((D, D)), _full((D, D))],
        out_specs=[pl.BlockSpec((R, D), lambda i: (i, 0))] * 3,
        out_shape=[out, out, out],
    )(x, Wa, ba, WmA0, WmB0)


def _edgefold(ea, Wb, bb, WmC0, bm0, WmC1, bm1):
    E, DE = ea.shape
    R = 4000
    def body(ea_r, Wb_r, bb_r, C0_r, b0_r, C1_r, b1_r, e0_r, e1_r):
        e = jnp.dot(ea_r[...], Wb_r[...], preferred_element_type=jnp.float32) + bb_r[...]
        e0_r[...] = jnp.dot(e, C0_r[...], preferred_element_type=jnp.float32) + b0_r[...]
        e1_r[...] = jnp.dot(e, C1_r[...], preferred_element_type=jnp.float32) + b1_r[...]
    out = jax.ShapeDtypeStruct((E, D), jnp.float32)
    return pl.pallas_call(
        body,
        grid=(E // R,),
        in_specs=[pl.BlockSpec((R, DE), lambda i: (i, 0)), _full((DE, DE)),
                  _full((1, DE)), _full((DE, D)), _full((1, D)),
                  _full((DE, D)), _full((1, D))],
        out_specs=[pl.BlockSpec((R, D), lambda i: (i, 0))] * 2,
        out_shape=[out, out],
    )(ea, Wb, bb, WmC0, bm0, WmC1, bm1)


def _msg(ga, gb, ee, gm, bem):
    E = ga.shape[0]
    R = 4000
    def body(ga_r, gb_r, ee_r, g_r, b_r, m_r):
        z = ga_r[...] + gb_r[...] + ee_r[...]
        m_r[...] = _ln_tanh(z, g_r[...], b_r[...])
    return pl.pallas_call(
        body,
        grid=(E // R,),
        in_specs=[pl.BlockSpec((R, D), lambda i: (i, 0))] * 3 + [_full((1, D))] * 2,
        out_specs=pl.BlockSpec((R, D), lambda i: (i, 0)),
        out_shape=jax.ShapeDtypeStruct((E, D), jnp.float32),
    )(ga, gb, ee, gm, bem)


def _update(P, h, WuA, WuB, bu, gu, beu, WmA, WmB):
    N = h.shape[0]
    R = 2000
    def body(P_r, h_r, WuA_r, WuB_r, bu_r, gu_r, beu_r, WmA_r, WmB_r,
             h1_r, a_r, b_r):
        aggr = P_r[0] + P_r[1]
        z = (jnp.dot(aggr, WuA_r[...], preferred_element_type=jnp.float32)
             + jnp.dot(h_r[...], WuB_r[...], preferred_element_type=jnp.float32)
             + bu_r[...])
        h1 = _ln_tanh(z, gu_r[...], beu_r[...])
        h1_r[...] = h1
        a_r[...] = jnp.dot(h1, WmA_r[...], preferred_element_type=jnp.float32)
        b_r[...] = jnp.dot(h1, WmB_r[...], preferred_element_type=jnp.float32)
    out = jax.ShapeDtypeStruct((N, D), jnp.float32)
    return pl.pallas_call(
        body,
        grid=(N // R,),
        in_specs=[pl.BlockSpec((2, R, D), lambda i: (0, i, 0)),
                  pl.BlockSpec((R, D), lambda i: (i, 0)),
                  _full((D, D)), _full((D, D)), _full((1, D)),
                  _full((1, D)), _full((1, D)), _full((D, D)), _full((D, D))],
        out_specs=[pl.BlockSpec((R, D), lambda i: (i, 0))] * 3,
        out_shape=[out, out, out],
    )(P, h, WuA, WuB, bu, gu, beu, WmA, WmB)


def _final(P, h, WuA, WuB, bu, gu, beu, W1, b1, W2, b2, batch2d):
    N = h.shape[0]
    R = 2000
    nblk = N // R
    def body(P_r, h_r, WuA_r, WuB_r, bu_r, gu_r, beu_r,
             W1_r, b1_r, W2_r, b2_r, bi_r, out_r, sums, cnts):
        i = pl.program_id(0)
        aggr = P_r[0] + P_r[1]
        z = (jnp.dot(aggr, WuA_r[...], preferred_element_type=jnp.float32)
             + jnp.dot(h_r[...], WuB_r[...], preferred_element_type=jnp.float32)
             + bu_r[...])
        h2 = _ln_tanh(z, gu_r[...], beu_r[...])
        hid = jax.nn.relu(jnp.dot(h2, W1_r[...], preferred_element_type=jnp.float32)
                          + b1_r[...])
        r = jnp.dot(hid, W2_r[...], preferred_element_type=jnp.float32) + b2_r[...]
        gids = jax.lax.broadcasted_iota(jnp.int32, (R, G), 1)
        mask = (bi_r[...] == gids).astype(jnp.float32)
        blk_sum = jax.lax.dot_general(
            r, mask, (((0,), (0,)), ((), ())), preferred_element_type=jnp.float32)
        blk_cnt = jnp.sum(mask, axis=0, keepdims=True)

        @pl.when(i == 0)
        def _():
            sums[...] = jnp.zeros_like(sums)
            cnts[...] = jnp.zeros_like(cnts)
        sums[...] += blk_sum
        cnts[...] += blk_cnt

        @pl.when(i == nblk - 1)
        def _():
            out_r[...] = sums[...] / jnp.maximum(cnts[...], 1.0)
    return pl.pallas_call(
        body,
        grid=(nblk,),
        in_specs=[pl.BlockSpec((2, R, D), lambda i: (0, i, 0)),
                  pl.BlockSpec((R, D), lambda i: (i, 0)),
                  _full((D, D)), _full((D, D)), _full((1, D)),
                  _full((1, D)), _full((1, D)),
                  _full((D, D)), _full((1, D)), _full((D, 1)), _full((1, 1)),
                  pl.BlockSpec((R, 1), lambda i: (i, 0))],
        out_specs=_full((1, G)),
        out_shape=jax.ShapeDtypeStruct((1, G), jnp.float32),
        scratch_shapes=[pltpu.VMEM((1, G), jnp.float32),
                        pltpu.VMEM((1, G), jnp.float32)],
    )(P, h, WuA, WuB, bu, gu, beu, W1, b1, W2, b2, batch2d)


def _sc_gather(A, B, src, dst):
    E = src.shape[0]
    n_chunks = E // CHUNK
    per = -(-n_chunks // NW)
    mesh = plsc.VectorSubcoreMesh(core_axis_name="c", subcore_axis_name="s",
                                  num_cores=NC, num_subcores=NS)
    out = jax.ShapeDtypeStruct((E, D), jnp.float32)

    @functools.partial(
        pl.kernel, out_type=(out, out), mesh=mesh,
        scratch_types=[
            pltpu.VMEM((CHUNK,), jnp.int32),
            pltpu.VMEM((CHUNK,), jnp.int32),
            pltpu.VMEM((CHUNK, D), jnp.float32),
            pltpu.VMEM((CHUNK, D), jnp.float32),
            pltpu.SemaphoreType.DMA,
            pltpu.SemaphoreType.DMA,
        ])
    def k(A_h, B_h, src_h, dst_h, GA_h, GB_h, idxs, idxd, bufa, bufb, s1, s2):
        wid = lax.axis_index("s") * NC + lax.axis_index("c")

        def body(g, carry):
            cid = g * NW + wid

            @pl.when(cid < n_chunks)
            def _():
                base = cid * CHUNK
                pltpu.sync_copy(src_h.at[pl.ds(base, CHUNK)], idxs)
                pltpu.sync_copy(dst_h.at[pl.ds(base, CHUNK)], idxd)
                ca = pltpu.async_copy(A_h.at[idxs], bufa, s1)
                cb = pltpu.async_copy(B_h.at[idxd], bufb, s2)
                ca.wait()
                cb.wait()
                pltpu.sync_copy(bufa, GA_h.at[pl.ds(base, CHUNK)])
                pltpu.sync_copy(bufb, GB_h.at[pl.ds(base, CHUNK)])
            return carry

        lax.fori_loop(0, per, body, 0)

    return k(A, B, src, dst)


def _sc_scatter(M, dst):
    E = dst.shape[0]
    N = 10000
    row_step = 624
    row_span = 640
    zrows = 128
    n_chunks = E // CHUNK
    per = -(-n_chunks // NW)
    mesh = plsc.VectorSubcoreMesh(core_axis_name="c", subcore_axis_name="s",
                                  num_cores=NC, num_subcores=NS)

    @functools.partial(
        pl.kernel, out_type=jax.ShapeDtypeStruct((NC, N, D), jnp.float32),
        mesh=mesh,
        scratch_types=[
            pltpu.VMEM_SHARED((N, D), jnp.float32),
            pltpu.VMEM((CHUNK, D), jnp.float32),
            pltpu.VMEM((CHUNK,), jnp.int32),
            pltpu.VMEM((zrows, D), jnp.float32),
        ])
    def k(M_h, dst_h, P_h, aggr, bufm, idxd, zbuf):
        c = lax.axis_index("c")
        s = lax.axis_index("s")
        wid = s * NC + c

        def zero_row(i, carry):
            for j in range(D // 16):
                zbuf[i, pl.ds(j * 16, 16)] = jnp.zeros((16,), jnp.float32)
            return carry

        lax.fori_loop(0, zrows, zero_row, 0)
        for kk in range(row_span // zrows):
            pltpu.sync_copy(zbuf, aggr.at[pl.ds(s * row_step + kk * zrows, zrows)])
        plsc.subcore_barrier()

        def body(g, carry):
            cid = g * NW + wid

            @pl.when(cid < n_chunks)
            def _():
                base = cid * CHUNK
                pltpu.sync_copy(dst_h.at[pl.ds(base, CHUNK)], idxd)
                pltpu.sync_copy(M_h.at[pl.ds(base, CHUNK)], bufm)
                pltpu.sync_copy(bufm, aggr.at[idxd], add=True)
            return carry

        lax.fori_loop(0, per, body, 0)
        plsc.subcore_barrier()
        pltpu.sync_copy(aggr.at[pl.ds(s * row_step, row_span)],
                        P_h.at[c, pl.ds(s * row_step, row_span)])

    return k(M, dst)


def kernel(x, edge_index, edge_attr, batch_idx, Wa, ba, Wb, bb,
           Wm0, bm0, gm0, betam0, Wu0, bu0, gu0, betau0,
           Wm1, bm1, gm1, betam1, Wu1, bu1, gu1, betau1,
           W1, b1, W2, b2):
    N = x.shape[0]
    src = edge_index[0]
    dst = edge_index[1]

    def row(v):
        return v.reshape(1, -1)

    h, A, B = _node0(x, Wa, row(ba), Wm0[:D], Wm0[D:2 * D])
    Ee0, Ee1 = _edgefold(edge_attr, Wb, row(bb),
                         Wm0[2 * D:], row(bm0), Wm1[2 * D:], row(bm1))

    GA, GB = _sc_gather(A, B, src, dst)
    M = _msg(GA, GB, Ee0, row(gm0), row(betam0))
    P = _sc_scatter(M, dst)
    h, A, B = _update(P, h, Wu0[:D], Wu0[D:], row(bu0), row(gu0), row(betau0),
                      Wm1[:D], Wm1[D:2 * D])

    GA, GB = _sc_gather(A, B, src, dst)
    M = _msg(GA, GB, Ee1, row(gm1), row(betam1))
    P = _sc_scatter(M, dst)

    return _final(P, h, Wu1[:D], Wu1[D:], row(bu1), row(gu1), row(betau1),
                  W1, row(b1), W2.reshape(D, 1), b2.reshape(1, 1),
                  batch_idx.reshape(N, 1))
